# baseline (speedup 1.0000x reference)
import jax
import jax.numpy as jnp
from jax import lax
from jax.experimental import pallas as pl
from jax.experimental.pallas import tpu as pltpu
from jax.experimental.pallas import tpu_sc as plsc

B = 16384
NWPAD = 1000448
WPIECE = 7816
F = 26
FP = 32
D = 16
N_FEAT = 1000100
NC = 2
NS = 16
NW = NC * NS
RPW = B // NW
C = 32
NCHUNK = RPW // C
EC = C * F
PC = C * FP
GSZ_E = EC
GSZ_W = PC


def _fm_body(idx26_hbm, idx32_hbm, val32_hbm, embed_hbm, w_hbm, b16_hbm,
             out_hbm,
             idx26_v0, idx26_v1, idx32_v0, idx32_v1, val_v0, val_v1,
             rows_v0, rows_v1, w_v0, w_v1, comb_v, out_v, b_v, bounce_v,
             w_sh, sem_g, sem_s):
    sid = lax.axis_index("s")
    wid = sid * NC + lax.axis_index("c")
    row0 = wid * RPW

    wbase = pl.multiple_of(sid * (NWPAD // NS), 8)
    for k in range(NWPAD // NS // WPIECE):
        off = pl.multiple_of(wbase + k * WPIECE, 8)
        pltpu.sync_copy(w_hbm.at[pl.ds(off, WPIECE)], bounce_v)
        pltpu.sync_copy(bounce_v, w_sh.at[pl.ds(off, WPIECE)])
    plsc.subcore_barrier()

    pltpu.sync_copy(b16_hbm, b_v)
    breg = b_v[...]

    idx26_b = (idx26_v0, idx26_v1)
    idx32_b = (idx32_v0, idx32_v1)
    val_b = (val_v0, val_v1)
    rows_b = (rows_v0, rows_v1)
    w_b = (w_v0, w_v1)

    def stage(i):
        p = i % 2
        e_base = pl.multiple_of((row0 + i * C) * F, 8)
        p_base = pl.multiple_of((row0 + i * C) * FP, 8)
        return [
            pltpu.async_copy(idx26_hbm.at[pl.ds(e_base, EC)], idx26_b[p], sem_s),
            pltpu.async_copy(idx32_hbm.at[pl.ds(p_base, PC)], idx32_b[p], sem_s),
            pltpu.async_copy(val32_hbm.at[pl.ds(p_base, PC)], val_b[p], sem_s),
        ]

    def fire(i):
        p = i % 2
        cps = []
        for k in range(EC // GSZ_E):
            cps.append(pltpu.async_copy(
                embed_hbm.at[idx26_b[p].at[pl.ds(k * GSZ_E, GSZ_E)]],
                rows_b[p].at[pl.ds(k * GSZ_E, GSZ_E), :], sem_g))
        for k in range(PC // GSZ_W):
            cps.append(pltpu.async_copy(
                w_hbm.at[idx32_b[p].at[pl.ds(k * GSZ_W, GSZ_W)]],
                w_b[p].at[pl.ds(k * GSZ_W, GSZ_W)], sem_g))
        return cps

    def compute(i):
        p = i % 2
        rows_v, val_v, w_v = rows_b[p], val_b[p], w_b[p]

        def row_body(r, rcarry):
            fb26 = r * F
            fb32 = r * FP
            v0 = val_v[pl.ds(fb32, D)]
            v1 = val_v[pl.ds(fb32 + D, D)]
            w0 = w_v[pl.ds(fb32, D)]
            w1 = w_v[pl.ds(fb32 + D, D)]
            s = jnp.zeros((D,), jnp.float32)
            q = jnp.zeros((D,), jnp.float32)
            for j in range(F):
                e = rows_v[fb26 + j]
                vj = v0[j] if j < D else v1[j - D]
                t = e * vj
                s = s + t
                q = q + t * t
            comb_v[pl.ds(r * D, D)] = 0.5 * (s * s - q) + v0 * w0 + v1 * w1 + breg
            return rcarry

        lax.fori_loop(0, C, row_body, 0)

        flat_iota = lax.iota(jnp.int32, D) * D
        for g in range(C // D):
            base = flat_iota + g * D * D
            acc = plsc.load_gather(comb_v, [base])
            for l in range(1, D):
                acc = acc + plsc.load_gather(comb_v, [base + l])
            out_v[pl.ds(i * C + g * D, D)] = acc

    st = stage(0)
    for cp in st:
        cp.wait()
    g = fire(0)
    for i in range(NCHUNK):
        if i + 1 < NCHUNK:
            st = stage(i + 1)
        for cp in g:
            cp.wait()
        if i + 1 < NCHUNK:
            for cp in st:
                cp.wait()
            g = fire(i + 1)
        compute(i)

    pltpu.sync_copy(out_v, out_hbm.at[pl.ds(pl.multiple_of(row0, 8), RPW)])


@jax.jit
def _fm(idx26, idx32, val32, feature_embed, linear_w, b16):
    fm = pl.kernel(
        _fm_body,
        out_type=jax.ShapeDtypeStruct((B,), jnp.float32),
        mesh=plsc.VectorSubcoreMesh(core_axis_name="c", subcore_axis_name="s"),
        compiler_params=pltpu.CompilerParams(
            needs_layout_passes=False, use_tc_tiling_on_sc=False),
        scratch_types=[
            pltpu.VMEM((EC,), jnp.int32),
            pltpu.VMEM((EC,), jnp.int32),
            pltpu.VMEM((PC,), jnp.int32),
            pltpu.VMEM((PC,), jnp.int32),
            pltpu.VMEM((PC,), jnp.float32),
            pltpu.VMEM((PC,), jnp.float32),
            pltpu.VMEM((EC, D), jnp.float32),
            pltpu.VMEM((EC, D), jnp.float32),
            pltpu.VMEM((PC,), jnp.float32),
            pltpu.VMEM((PC,), jnp.float32),
            pltpu.VMEM((C * D,), jnp.float32),
            pltpu.VMEM((RPW,), jnp.float32),
            pltpu.VMEM((D,), jnp.float32),
            pltpu.VMEM((WPIECE,), jnp.float32),
            pltpu.VMEM_SHARED((NWPAD,), jnp.float32),
            pltpu.SemaphoreType.DMA,
            pltpu.SemaphoreType.DMA,
        ],
    )
    return fm(idx26, idx32, val32, feature_embed, linear_w, b16)


def kernel(feat_idx, feat_val, feature_embed, linear_w, linear_b):
    idx26 = feat_idx.reshape(-1).astype(jnp.int32)
    idx32 = jnp.pad(feat_idx.astype(jnp.int32), ((0, 0), (0, FP - F))).reshape(-1)
    val32 = jnp.pad(feat_val, ((0, 0), (0, FP - F))).reshape(-1)
    wpad = jnp.pad(linear_w, (0, NWPAD - N_FEAT))
    b16 = jnp.full((D,), linear_b / D, dtype=jnp.float32)
    return _fm(idx26, idx32, val32, feature_embed, wpad, b16)


# speedup vs baseline: 1.9828x; 1.9828x over previous
import jax
import jax.numpy as jnp
from jax import lax
from jax.experimental import pallas as pl
from jax.experimental.pallas import tpu as pltpu
from jax.experimental.pallas import tpu_sc as plsc

B = 16384
F = 26
FP = 32
D = 16
NC = 2
NS = 16
NW = NC * NS
RPW = B // NW
C = 32
NCHUNK = RPW // C
EC = C * F
PC = C * FP


def _fm_body(idx26_hbm, val32_hbm, embed_hbm, w_hbm, b16_hbm, out_hbm,
             idx26_v0, idx26_v1, val_v0, val_v1, rows_v0, rows_v1,
             w26_v0, w26_v1, w_v, comb_v, out_v, b_v, sem_g, sem_s):
    sid = lax.axis_index("s")
    wid = sid * NC + lax.axis_index("c")
    row0 = wid * RPW

    pltpu.sync_copy(b16_hbm, b_v)
    breg = b_v[...]

    idx26_b = (idx26_v0, idx26_v1)
    val_b = (val_v0, val_v1)
    rows_b = (rows_v0, rows_v1)
    w26_b = (w26_v0, w26_v1)

    zero16 = jnp.zeros((D,), jnp.float32)
    w26_v0[pl.ds(EC, D)] = zero16
    w26_v1[pl.ds(EC, D)] = zero16

    def stage(i):
        p = i % 2
        e_base = pl.multiple_of((row0 + i * C) * F, 8)
        p_base = pl.multiple_of((row0 + i * C) * FP, 8)
        return [
            pltpu.async_copy(idx26_hbm.at[pl.ds(e_base, EC)], idx26_b[p], sem_s),
            pltpu.async_copy(val32_hbm.at[pl.ds(p_base, PC)], val_b[p], sem_s),
        ]

    def fire(i):
        p = i % 2
        return [
            pltpu.async_copy(
                embed_hbm.at[idx26_b[p]], rows_b[p], sem_g),
            pltpu.async_copy(
                w_hbm.at[idx26_b[p]], w26_b[p].at[pl.ds(0, EC)], sem_g),
        ]

    def compute(i):
        p = i % 2
        rows_v, val_v, w26_v = rows_b[p], val_b[p], w26_b[p]

        for g in range(PC // D):
            off = (g // 2) * F + (g % 2) * D
            w_v[pl.ds(g * D, D)] = w26_v[pl.ds(off, D)]

        def row_body(r, rcarry):
            fb26 = r * F
            fb32 = r * FP
            v0 = val_v[pl.ds(fb32, D)]
            v1 = val_v[pl.ds(fb32 + D, D)]
            w0 = w_v[pl.ds(fb32, D)]
            w1 = w_v[pl.ds(fb32 + D, D)]
            s = jnp.zeros((D,), jnp.float32)
            q = jnp.zeros((D,), jnp.float32)
            for j in range(F):
                e = rows_v[fb26 + j]
                vj = v0[j] if j < D else v1[j - D]
                t = e * vj
                s = s + t
                q = q + t * t
            comb_v[pl.ds(r * D, D)] = 0.5 * (s * s - q) + v0 * w0 + v1 * w1 + breg
            return rcarry

        lax.fori_loop(0, C, row_body, 0)

        flat_iota = lax.iota(jnp.int32, D) * D
        for g in range(C // D):
            base = flat_iota + g * D * D
            acc = plsc.load_gather(comb_v, [base])
            for l in range(1, D):
                acc = acc + plsc.load_gather(comb_v, [base + l])
            out_v[pl.ds(i * C + g * D, D)] = acc

    st = stage(0)
    for cp in st:
        cp.wait()
    g = fire(0)
    for i in range(NCHUNK):
        if i + 1 < NCHUNK:
            st = stage(i + 1)
        for cp in g:
            cp.wait()
        if i + 1 < NCHUNK:
            for cp in st:
                cp.wait()
            g = fire(i + 1)
        compute(i)

    pltpu.sync_copy(out_v, out_hbm.at[pl.ds(pl.multiple_of(row0, 8), RPW)])


@jax.jit
def _fm(idx26, val32, feature_embed, linear_w, b16):
    fm = pl.kernel(
        _fm_body,
        out_type=jax.ShapeDtypeStruct((B,), jnp.float32),
        mesh=plsc.VectorSubcoreMesh(core_axis_name="c", subcore_axis_name="s"),
        compiler_params=pltpu.CompilerParams(
            needs_layout_passes=False, use_tc_tiling_on_sc=False),
        scratch_types=[
            pltpu.VMEM((EC,), jnp.int32),
            pltpu.VMEM((EC,), jnp.int32),
            pltpu.VMEM((PC,), jnp.float32),
            pltpu.VMEM((PC,), jnp.float32),
            pltpu.VMEM((EC, D), jnp.float32),
            pltpu.VMEM((EC, D), jnp.float32),
            pltpu.VMEM((EC + D,), jnp.float32),
            pltpu.VMEM((EC + D,), jnp.float32),
            pltpu.VMEM((PC,), jnp.float32),
            pltpu.VMEM((C * D,), jnp.float32),
            pltpu.VMEM((RPW,), jnp.float32),
            pltpu.VMEM((D,), jnp.float32),
            pltpu.SemaphoreType.DMA,
            pltpu.SemaphoreType.DMA,
        ],
    )
    return fm(idx26, val32, feature_embed, linear_w, b16)


def kernel(feat_idx, feat_val, feature_embed, linear_w, linear_b):
    idx26 = feat_idx.reshape(-1).astype(jnp.int32)
    val32 = jnp.pad(feat_val, ((0, 0), (0, FP - F))).reshape(-1)
    b16 = jnp.full((D,), linear_b / D, dtype=jnp.float32)
    return _fm(idx26, val32, feature_embed, linear_w, b16)
